# baseline (speedup 1.0000x reference)
import functools

import jax
import jax.numpy as jnp
from jax import lax
from jax.experimental import pallas as pl
from jax.experimental.pallas import tpu as pltpu
from jax.experimental.pallas import tpu_sc as plsc

_C = 128
_HW = 512 * 512
_K = 65536
_NCLS = 10
_NPAD = 16
_S = 4096


def _classify_block(feat_ref, wpbig_ref, bp_ref, bmask_ref, out_ref):
    f = feat_ref[...].reshape(_C * 8, 512)
    packed = lax.dot_general(
        f, wpbig_ref[...], (((0,), (0,)), ((), ())),
        preferred_element_type=jnp.float32)
    packed = packed + bp_ref[...]
    m = jnp.max(packed, axis=1, keepdims=True)
    e = jnp.exp(packed - m)
    s = lax.dot_general(
        e, bmask_ref[...], (((1,), (0,)), ((), ())),
        precision=lax.Precision.HIGHEST,
        preferred_element_type=jnp.float32)
    out_ref[...] = e / s


def _softmax_table(feat, wpbig, bp, bmask):
    return pl.pallas_call(
        _classify_block,
        grid=(_HW // _S,),
        in_specs=[
            pl.BlockSpec((1, _C, 8, 512), lambda i: (0, 0, i, 0)),
            pl.BlockSpec((_C * 8, 128), lambda i: (0, 0)),
            pl.BlockSpec((1, 128), lambda i: (0, 0)),
            pl.BlockSpec((128, 128), lambda i: (0, 0)),
        ],
        out_specs=pl.BlockSpec((_S // 8, 128), lambda i: (i, 0)),
        out_shape=jax.ShapeDtypeStruct((_HW // 8, 128), jnp.float32),
        compiler_params=pltpu.CompilerParams(
            dimension_semantics=("arbitrary",)),
    )(feat, wpbig, bp, bmask)


def _make_row_gather():
    info = plsc.get_sparse_core_info()
    nc, ns = info.num_cores, info.num_subcores
    bpw = _K // (nc * ns)
    ch = 512
    mesh = plsc.VectorSubcoreMesh(core_axis_name="c", subcore_axis_name="s")

    @functools.partial(
        pl.kernel, mesh=mesh,
        out_type=jax.ShapeDtypeStruct((_K, _NPAD), jnp.float32),
        scratch_types=[
            pltpu.VMEM((bpw,), jnp.int32),
            pltpu.VMEM((bpw,), jnp.int32),
            pltpu.VMEM((bpw, _NPAD), jnp.float32),
            pltpu.SemaphoreType.DMA,
        ],
        compiler_params=pltpu.CompilerParams(use_tc_tiling_on_sc=False),
    )
    def gather_rows(table_hbm, idx_hbm, out_hbm, idx_v, idx2_v, rows_v, sem):
        wid = lax.axis_index("s") * nc + lax.axis_index("c")
        base = wid * bpw
        pltpu.sync_copy(idx_hbm.at[0, pl.ds(base, bpw)], idx_v)

        def remap(j, carry):
            v = idx_v[pl.ds(j * 16, 16)]
            idx2_v[pl.ds(j * 16, 16)] = (
                (v & ~4095) | ((v & 511) << 3) | ((v >> 9) & 7))
            return carry

        lax.fori_loop(0, bpw // 16, remap, 0)
        pltpu.async_copy(table_hbm.at[idx2_v], rows_v, sem).wait()
        pltpu.sync_copy(rows_v, out_hbm.at[pl.ds(base, bpw)])

    return gather_rows


def kernel(gc_features, cls_id_map, W, b):
    wp = jnp.zeros((_NPAD, _C), jnp.float32).at[:_NCLS, :].set(W).T
    wpbig = (wp[:, None, None, :] * jnp.eye(8, dtype=jnp.float32)[None, :, :, None]
             ).reshape(_C * 8, 128)
    bpad = jnp.full((_NPAD,), -1e30, jnp.float32).at[:_NCLS].set(b)
    bp = jnp.tile(bpad, 8).reshape(1, 128)
    bmask = jnp.kron(jnp.eye(8, dtype=jnp.float32),
                     jnp.ones((_NPAD, _NPAD), jnp.float32))
    table = _softmax_table(gc_features, wpbig, bp, bmask).reshape(_HW, _NPAD)
    rows = _make_row_gather()(table, cls_id_map)
    return rows[:, :_NCLS]


# speedup vs baseline: 1.0957x; 1.0957x over previous
import functools

import jax
import jax.numpy as jnp
from jax import lax
from jax.experimental import pallas as pl
from jax.experimental.pallas import tpu as pltpu
from jax.experimental.pallas import tpu_sc as plsc

_C = 128
_HW = 512 * 512
_K = 65536
_NCLS = 10
_NPAD = 16
_S = 4096


def _classify_block(feat_ref, wpbig_ref, bp_ref, bmask_ref, out_ref):
    f = feat_ref[...].reshape(_C * 8, 512)
    packed = lax.dot_general(
        f, wpbig_ref[...], (((0,), (0,)), ((), ())),
        preferred_element_type=jnp.float32)
    packed = packed + bp_ref[...]
    m = jnp.max(packed, axis=1, keepdims=True)
    e = jnp.exp(packed - m)
    s = lax.dot_general(
        e, bmask_ref[...], (((1,), (0,)), ((), ())),
        precision=lax.Precision.HIGHEST,
        preferred_element_type=jnp.float32)
    out_ref[...] = e / s


def _softmax_table(feat, wpbig, bp, bmask):
    return pl.pallas_call(
        _classify_block,
        grid=(_HW // _S,),
        in_specs=[
            pl.BlockSpec((1, _C, 8, 512), lambda i: (0, 0, i, 0)),
            pl.BlockSpec((_C * 8, 128), lambda i: (0, 0)),
            pl.BlockSpec((1, 128), lambda i: (0, 0)),
            pl.BlockSpec((128, 128), lambda i: (0, 0)),
        ],
        out_specs=pl.BlockSpec((_S // 8, 128), lambda i: (i, 0)),
        out_shape=jax.ShapeDtypeStruct((_HW // 8, 128), jnp.float32),
        compiler_params=pltpu.CompilerParams(
            dimension_semantics=("arbitrary",)),
    )(feat, wpbig, bp, bmask)


def _make_row_gather():
    info = plsc.get_sparse_core_info()
    nc, ns = info.num_cores, info.num_subcores
    bpw = _K // (nc * ns)
    ch = 512
    mesh = plsc.VectorSubcoreMesh(core_axis_name="c", subcore_axis_name="s")

    @functools.partial(
        pl.kernel, mesh=mesh,
        out_type=jax.ShapeDtypeStruct((_NPAD, _K), jnp.float32),
        scratch_types=[
            pltpu.VMEM((bpw,), jnp.int32),
            pltpu.VMEM((bpw,), jnp.int32),
            pltpu.VMEM((bpw, _NPAD), jnp.float32),
            pltpu.VMEM((_NPAD * bpw,), jnp.float32),
            pltpu.SemaphoreType.DMA,
        ],
        compiler_params=pltpu.CompilerParams(
            use_tc_tiling_on_sc=False, needs_layout_passes=False),
    )
    def gather_rows(table_hbm, idx_hbm, out_hbm, idx_v, idx2_v, rows_v,
                    rowsT_v, sem):
        wid = lax.axis_index("s") * nc + lax.axis_index("c")
        base = wid * bpw
        pltpu.sync_copy(idx_hbm.at[0, pl.ds(base, bpw)], idx_v)

        def remap(j, carry):
            v = idx_v[pl.ds(j * 16, 16)]
            idx2_v[pl.ds(j * 16, 16)] = (
                (v & ~4095) | ((v & 511) << 3) | ((v >> 9) & 7))
            return carry

        lax.fori_loop(0, bpw // 16, remap, 0)
        pltpu.async_copy(table_hbm.at[idx2_v], rows_v, sem).wait()

        strides = lax.iota(jnp.int32, 16) * bpw

        def tr(j, carry):
            plsc.store_scatter(rowsT_v, [strides + j], rows_v[j, :])
            return carry

        lax.fori_loop(0, bpw, tr, 0)
        for c in range(_NPAD):
            pltpu.sync_copy(rowsT_v.at[pl.ds(c * bpw, bpw)],
                            out_hbm.at[c, pl.ds(base, bpw)])

    return gather_rows


def kernel(gc_features, cls_id_map, W, b):
    wp = jnp.zeros((_NPAD, _C), jnp.float32).at[:_NCLS, :].set(W).T
    wpbig = (wp[:, None, None, :] * jnp.eye(8, dtype=jnp.float32)[None, :, :, None]
             ).reshape(_C * 8, 128)
    bpad = jnp.full((_NPAD,), -1e30, jnp.float32).at[:_NCLS].set(b)
    bp = jnp.tile(bpad, 8).reshape(1, 128)
    bmask = jnp.kron(jnp.eye(8, dtype=jnp.float32),
                     jnp.ones((_NPAD, _NPAD), jnp.float32))
    table = _softmax_table(gc_features, wpbig, bp, bmask).reshape(_HW, _NPAD)
    rowsT = _make_row_gather()(table, cls_id_map)
    return rowsT[:_NCLS, :].T


# speedup vs baseline: 1.2782x; 1.1666x over previous
import functools

import jax
import jax.numpy as jnp
from jax import lax
from jax.experimental import pallas as pl
from jax.experimental.pallas import tpu as pltpu
from jax.experimental.pallas import tpu_sc as plsc

_C = 128
_HW = 512 * 512
_K = 65536
_NCLS = 10
_NPAD = 16
_S = 4096


def _classify_block(feat_ref, wpbig_ref, bp_ref, bmask_ref, out_ref):
    x = feat_ref[...]
    halves = []
    for h in range(2):
        f = x[0, :, 8 * h:8 * h + 8, :].reshape(_C * 8, 512)
        halves.append(lax.dot_general(
            f, wpbig_ref[...], (((0,), (0,)), ((), ())),
            preferred_element_type=jnp.float32))
    packed = jnp.concatenate(halves, axis=0)
    packed = packed + bp_ref[...]
    m = jnp.max(packed, axis=1, keepdims=True)
    e = jnp.exp(packed - m)
    s = lax.dot_general(
        e, bmask_ref[...], (((1,), (0,)), ((), ())),
        precision=lax.Precision.HIGHEST,
        preferred_element_type=jnp.float32)
    out_ref[...] = e / s


def _softmax_table(feat, wpbig, bp, bmask):
    return pl.pallas_call(
        _classify_block,
        grid=(_HW // (2 * _S),),
        in_specs=[
            pl.BlockSpec((1, _C, 16, 512), lambda i: (0, 0, i, 0)),
            pl.BlockSpec((_C * 8, 128), lambda i: (0, 0)),
            pl.BlockSpec((1, 128), lambda i: (0, 0)),
            pl.BlockSpec((128, 128), lambda i: (0, 0)),
        ],
        out_specs=pl.BlockSpec((_S // 4, 128), lambda i: (i, 0)),
        out_shape=jax.ShapeDtypeStruct((_HW // 8, 128), jnp.float32),
        compiler_params=pltpu.CompilerParams(
            dimension_semantics=("arbitrary",)),
    )(feat, wpbig, bp, bmask)


def _make_row_gather():
    info = plsc.get_sparse_core_info()
    nc, ns = info.num_cores, info.num_subcores
    bpw = _K // (nc * ns)
    ch = 512
    mesh = plsc.VectorSubcoreMesh(core_axis_name="c", subcore_axis_name="s")

    @functools.partial(
        pl.kernel, mesh=mesh,
        out_type=jax.ShapeDtypeStruct((_NPAD, _K), jnp.float32),
        scratch_types=[
            pltpu.VMEM((bpw,), jnp.int32),
            pltpu.VMEM((bpw,), jnp.int32),
            pltpu.VMEM((bpw, _NPAD), jnp.float32),
            pltpu.VMEM((_NPAD * bpw,), jnp.float32),
            pltpu.SemaphoreType.DMA,
        ],
        compiler_params=pltpu.CompilerParams(
            use_tc_tiling_on_sc=False, needs_layout_passes=False),
    )
    def gather_rows(table_hbm, idx_hbm, out_hbm, idx_v, idx2_v, rows_v,
                    rowsT_v, sem):
        wid = lax.axis_index("s") * nc + lax.axis_index("c")
        base = wid * bpw
        pltpu.sync_copy(idx_hbm.at[0, pl.ds(base, bpw)], idx_v)

        def remap(j, carry):
            v = idx_v[pl.ds(j * 16, 16)]
            idx2_v[pl.ds(j * 16, 16)] = (
                (v & ~4095) | ((v & 511) << 3) | ((v >> 9) & 7))
            return carry

        lax.fori_loop(0, bpw // 16, remap, 0)
        pltpu.async_copy(table_hbm.at[idx2_v], rows_v, sem).wait()

        strides = lax.iota(jnp.int32, 16) * bpw

        def tr(j, carry):
            plsc.store_scatter(rowsT_v, [strides + j], rows_v[j, :])
            return carry

        lax.fori_loop(0, bpw, tr, 0)
        for c in range(_NPAD):
            pltpu.sync_copy(rowsT_v.at[pl.ds(c * bpw, bpw)],
                            out_hbm.at[c, pl.ds(base, bpw)])

    return gather_rows


def kernel(gc_features, cls_id_map, W, b):
    wp = jnp.zeros((_NPAD, _C), jnp.float32).at[:_NCLS, :].set(W).T
    wpbig = (wp[:, None, None, :] * jnp.eye(8, dtype=jnp.float32)[None, :, :, None]
             ).reshape(_C * 8, 128)
    bpad = jnp.full((_NPAD,), -1e30, jnp.float32).at[:_NCLS].set(b)
    bp = jnp.tile(bpad, 8).reshape(1, 128)
    bmask = jnp.kron(jnp.eye(8, dtype=jnp.float32),
                     jnp.ones((_NPAD, _NPAD), jnp.float32))
    table = _softmax_table(gc_features, wpbig, bp, bmask).reshape(_HW, _NPAD)
    rowsT = _make_row_gather()(table, cls_id_map)
    return rowsT[:_NCLS, :].T


# speedup vs baseline: 1.5173x; 1.1870x over previous
import functools

import jax
import jax.numpy as jnp
from jax import lax
from jax.experimental import pallas as pl
from jax.experimental.pallas import tpu as pltpu
from jax.experimental.pallas import tpu_sc as plsc

_C = 128
_HW = 512 * 512
_K = 65536
_NCLS = 10
_NPAD = 16
_ROWS = 32


def _classify_block(feat_ref, wpbig_ref, bp_ref, bmask_ref, out_ref):
    x = feat_ref[...]
    halves = []
    for h in range(_ROWS // 8):
        f = x[0, :, 8 * h:8 * h + 8, :].reshape(_C * 8, 512)
        halves.append(lax.dot_general(
            f, wpbig_ref[...], (((0,), (0,)), ((), ())),
            preferred_element_type=jnp.float32))
    packed = jnp.concatenate(halves, axis=0)
    packed = packed + bp_ref[...]
    m = jnp.max(packed, axis=1, keepdims=True)
    e = jnp.exp(packed - m)
    s = lax.dot_general(
        e, bmask_ref[...], (((1,), (0,)), ((), ())),
        precision=lax.Precision.HIGHEST,
        preferred_element_type=jnp.float32)
    out_ref[...] = e / s


def _softmax_table(feat, wpbig, bp, bmask):
    return pl.pallas_call(
        _classify_block,
        grid=(512 // _ROWS,),
        in_specs=[
            pl.BlockSpec((1, _C, _ROWS, 512), lambda i: (0, 0, i, 0)),
            pl.BlockSpec((_C * 8, 128), lambda i: (0, 0)),
            pl.BlockSpec((1, 128), lambda i: (0, 0)),
            pl.BlockSpec((128, 128), lambda i: (0, 0)),
        ],
        out_specs=pl.BlockSpec((_ROWS * 64, 128), lambda i: (i, 0)),
        out_shape=jax.ShapeDtypeStruct((_HW // 8, 128), jnp.float32),
        compiler_params=pltpu.CompilerParams(
            dimension_semantics=("arbitrary",)),
    )(feat, wpbig, bp, bmask)


def _make_row_gather():
    info = plsc.get_sparse_core_info()
    nc, ns = info.num_cores, info.num_subcores
    bpw = _K // (nc * ns)
    ch = 512
    mesh = plsc.VectorSubcoreMesh(core_axis_name="c", subcore_axis_name="s")

    @functools.partial(
        pl.kernel, mesh=mesh,
        out_type=jax.ShapeDtypeStruct((_NPAD, _K), jnp.float32),
        scratch_types=[
            pltpu.VMEM((bpw,), jnp.int32),
            pltpu.VMEM((bpw,), jnp.int32),
            pltpu.VMEM((bpw, _NPAD), jnp.float32),
            pltpu.VMEM((_NPAD * bpw,), jnp.float32),
            pltpu.SemaphoreType.DMA,
        ],
        compiler_params=pltpu.CompilerParams(
            use_tc_tiling_on_sc=False, needs_layout_passes=False),
    )
    def gather_rows(table_hbm, idx_hbm, out_hbm, idx_v, idx2_v, rows_v,
                    rowsT_v, sem):
        wid = lax.axis_index("s") * nc + lax.axis_index("c")
        base = wid * bpw
        pltpu.sync_copy(idx_hbm.at[0, pl.ds(base, bpw)], idx_v)

        def remap(j, carry):
            v = idx_v[pl.ds(j * 16, 16)]
            idx2_v[pl.ds(j * 16, 16)] = (
                (v & ~4095) | ((v & 511) << 3) | ((v >> 9) & 7))
            return carry

        lax.fori_loop(0, bpw // 16, remap, 0)
        pltpu.async_copy(table_hbm.at[idx2_v], rows_v, sem).wait()

        strides = lax.iota(jnp.int32, 16) * bpw

        def tr(j, carry):
            plsc.store_scatter(rowsT_v, [strides + j], rows_v[j, :])
            return carry

        lax.fori_loop(0, bpw, tr, 0)
        for c in range(_NPAD):
            pltpu.sync_copy(rowsT_v.at[pl.ds(c * bpw, bpw)],
                            out_hbm.at[c, pl.ds(base, bpw)])

    return gather_rows


def kernel(gc_features, cls_id_map, W, b):
    wp = jnp.zeros((_NPAD, _C), jnp.float32).at[:_NCLS, :].set(W).T
    wpbig = (wp[:, None, None, :] * jnp.eye(8, dtype=jnp.float32)[None, :, :, None]
             ).reshape(_C * 8, 128)
    bpad = jnp.full((_NPAD,), -1e30, jnp.float32).at[:_NCLS].set(b)
    bp = jnp.tile(bpad, 8).reshape(1, 128)
    bmask = jnp.kron(jnp.eye(8, dtype=jnp.float32),
                     jnp.ones((_NPAD, _NPAD), jnp.float32))
    table = _softmax_table(gc_features, wpbig, bp, bmask).reshape(_HW, _NPAD)
    rowsT = _make_row_gather()(table, cls_id_map)
    return rowsT[:_NCLS, :].T


# speedup vs baseline: 1.5721x; 1.0361x over previous
import functools

import jax
import jax.numpy as jnp
from jax import lax
from jax.experimental import pallas as pl
from jax.experimental.pallas import tpu as pltpu
from jax.experimental.pallas import tpu_sc as plsc

_C = 128
_HW = 512 * 512
_K = 65536
_NCLS = 10
_NPAD = 16
_ROWS = 64


def _classify_block(feat_ref, wpbig_ref, bp_ref, bmask_ref, out_ref):
    x = feat_ref[...]
    halves = []
    for h in range(_ROWS // 8):
        f = x[0, :, 8 * h:8 * h + 8, :].reshape(_C * 8, 512)
        halves.append(lax.dot_general(
            f, wpbig_ref[...], (((0,), (0,)), ((), ())),
            preferred_element_type=jnp.float32))
    packed = jnp.concatenate(halves, axis=0)
    packed = packed + bp_ref[...]
    m = jnp.max(packed, axis=1, keepdims=True)
    e = jnp.exp(packed - m)
    s = lax.dot_general(
        e, bmask_ref[...], (((1,), (0,)), ((), ())),
        precision=lax.Precision.HIGHEST,
        preferred_element_type=jnp.float32)
    out_ref[...] = e / s


def _softmax_table(feat, wpbig, bp, bmask):
    return pl.pallas_call(
        _classify_block,
        grid=(512 // _ROWS,),
        in_specs=[
            pl.BlockSpec((1, _C, _ROWS, 512), lambda i: (0, 0, i, 0)),
            pl.BlockSpec((_C * 8, 128), lambda i: (0, 0)),
            pl.BlockSpec((1, 128), lambda i: (0, 0)),
            pl.BlockSpec((128, 128), lambda i: (0, 0)),
        ],
        out_specs=pl.BlockSpec((_ROWS * 64, 128), lambda i: (i, 0)),
        out_shape=jax.ShapeDtypeStruct((_HW // 8, 128), jnp.float32),
        compiler_params=pltpu.CompilerParams(
            dimension_semantics=("arbitrary",)),
    )(feat, wpbig, bp, bmask)


def _make_row_gather():
    info = plsc.get_sparse_core_info()
    nc, ns = info.num_cores, info.num_subcores
    bpw = _K // (nc * ns)
    ch = 512
    mesh = plsc.VectorSubcoreMesh(core_axis_name="c", subcore_axis_name="s")

    @functools.partial(
        pl.kernel, mesh=mesh,
        out_type=jax.ShapeDtypeStruct((_NPAD, _K), jnp.float32),
        scratch_types=[
            pltpu.VMEM((bpw,), jnp.int32),
            pltpu.VMEM((bpw,), jnp.int32),
            pltpu.VMEM((bpw, _NPAD), jnp.float32),
            pltpu.VMEM((_NPAD * bpw,), jnp.float32),
            pltpu.SemaphoreType.DMA,
        ],
        compiler_params=pltpu.CompilerParams(
            use_tc_tiling_on_sc=False, needs_layout_passes=False),
    )
    def gather_rows(table_hbm, idx_hbm, out_hbm, idx_v, idx2_v, rows_v,
                    rowsT_v, sem):
        wid = lax.axis_index("s") * nc + lax.axis_index("c")
        base = wid * bpw
        pltpu.sync_copy(idx_hbm.at[0, pl.ds(base, bpw)], idx_v)

        def remap(j, carry):
            v = idx_v[pl.ds(j * 16, 16)]
            idx2_v[pl.ds(j * 16, 16)] = (
                (v & ~4095) | ((v & 511) << 3) | ((v >> 9) & 7))
            return carry

        lax.fori_loop(0, bpw // 16, remap, 0)
        pltpu.async_copy(table_hbm.at[idx2_v], rows_v, sem).wait()

        strides = lax.iota(jnp.int32, 16) * bpw

        def tr(j, carry):
            plsc.store_scatter(rowsT_v, [strides + j], rows_v[j, :])
            return carry

        lax.fori_loop(0, bpw, tr, 0)
        for c in range(_NPAD):
            pltpu.sync_copy(rowsT_v.at[pl.ds(c * bpw, bpw)],
                            out_hbm.at[c, pl.ds(base, bpw)])

    return gather_rows


def kernel(gc_features, cls_id_map, W, b):
    wp = jnp.zeros((_NPAD, _C), jnp.float32).at[:_NCLS, :].set(W).T
    wpbig = (wp[:, None, None, :] * jnp.eye(8, dtype=jnp.float32)[None, :, :, None]
             ).reshape(_C * 8, 128)
    bpad = jnp.full((_NPAD,), -1e30, jnp.float32).at[:_NCLS].set(b)
    bp = jnp.tile(bpad, 8).reshape(1, 128)
    bmask = jnp.kron(jnp.eye(8, dtype=jnp.float32),
                     jnp.ones((_NPAD, _NPAD), jnp.float32))
    table = _softmax_table(gc_features, wpbig, bp, bmask).reshape(_HW, _NPAD)
    rowsT = _make_row_gather()(table, cls_id_map)
    return rowsT[:_NCLS, :].T
